# baseline (speedup 1.0000x reference)
import functools

import jax
import jax.numpy as jnp
from jax import lax
from jax.experimental import pallas as pl
from jax.experimental.pallas import tpu as pltpu
from jax.experimental.pallas import tpu_sc as plsc

_N = 10000
_E = 160000
_D = 256
_DH = _D // 2
_NT = 16
_B = 96
_EPTP = 10368
_NBAT = _EPTP // _B
_CH = 3
_NCHUNK = _NBAT // _CH
_CB = _CH * _B
_MM = _NCHUNK // 3
_NBUF = 3
_NP = 10240
_RPT = _NP // _NT


def _sc_aggregate(x2, srcp, dstp, normp):
    mesh = plsc.VectorSubcoreMesh(core_axis_name="c", subcore_axis_name="s")

    @functools.partial(
        pl.kernel,
        mesh=mesh,
        out_type=jax.ShapeDtypeStruct((2, _NP, _DH), jnp.float32),
        scratch_types=(
            [pltpu.VMEM((_CB,), jnp.int32)] * 3
            + [pltpu.VMEM((_CB,), jnp.int32)] * 3
            + [pltpu.VMEM((_CB,), jnp.float32)] * 3
            + [pltpu.VMEM((_B,), jnp.int32)] * 3
            + [pltpu.VMEM((_B,), jnp.int32)] * 3
            + [pltpu.VMEM((_B, _DH), jnp.float32)] * 3
            + [pltpu.VMEM_SHARED((_NP, _DH), jnp.float32)]
            + [pltpu.SemaphoreType.DMA] * 8
        ),
    )
    def body(x2_hbm, src_hbm, dst_hbm, norm_hbm, out_hbm,
             sr0, sr1, sr2, dr0, dr1, dr2, nr0, nr1, nr2,
             di0, di1, di2, gi0, gi1, gi2, rw0, rw1, rw2, acc,
             ms0, ms1, ms2, gs0, gs1, gs2, ssem, zsem):
        src_rs = [sr0, sr1, sr2]
        dst_rs = [dr0, dr1, dr2]
        norm_rs = [nr0, nr1, nr2]
        didxs = [di0, di1, di2]
        gidxs = [gi0, gi1, gi2]
        rowss = [rw0, rw1, rw2]
        msems = [ms0, ms1, ms2]
        gsems = [gs0, gs1, gs2]
        c = lax.axis_index("c")
        s = lax.axis_index("s")
        r0 = s * _RPT
        e0 = s * _EPTP

        def meta_fetch(j, slot):
            off = e0 + j * _CB
            pltpu.async_copy(src_hbm.at[pl.ds(off, _CB)], src_rs[slot],
                             msems[slot])
            pltpu.async_copy(dst_hbm.at[pl.ds(off, _CB)], dst_rs[slot],
                             msems[slot])
            pltpu.async_copy(norm_hbm.at[pl.ds(off, _CB)], norm_rs[slot],
                             msems[slot])

        def meta_wait(j, slot):
            off = e0 + j * _CB
            pltpu.make_async_copy(src_hbm.at[pl.ds(off, _CB)],
                                  src_rs[slot], msems[slot]).wait()
            pltpu.make_async_copy(dst_hbm.at[pl.ds(off, _CB)],
                                  dst_rs[slot], msems[slot]).wait()
            pltpu.make_async_copy(norm_hbm.at[pl.ds(off, _CB)],
                                  norm_rs[slot], msems[slot]).wait()

        def start_gather(ch, bb, bg):
            for i in range(_B // 16):
                sl = pl.ds(i * 16, 16)
                gidxs[bg][sl] = src_rs[ch][pl.ds(bb * _B + i * 16, 16)] * 2 + c
            pltpu.async_copy(x2_hbm.at[gidxs[bg]], rowss[bg], gsems[bg])

        def wait_gather(b):
            pltpu.make_async_copy(x2_hbm.at[gidxs[b]], rowss[b],
                                  gsems[b]).wait()

        def wait_scatter(b):
            pltpu.make_async_copy(rowss[b], acc.at[didxs[b]], ssem).wait()

        def scale(b, q):
            def grp(g, inner):
                nv = norm_rs[q][pl.ds(b * _B + g * 16, 16)]
                for p in range(16):
                    e = g * 16 + p
                    nb = nv[p]
                    for j in range(_DH // 16):
                        sl = pl.ds(j * 16, 16)
                        rowss[b][e, sl] = rowss[b][e, sl] * nb
                return inner
            lax.fori_loop(0, _B // 16, grp, 0)

        meta_fetch(0, 0)
        meta_fetch(1, 1)
        def zero_grp(e, carry):
            for j in range(_DH // 16):
                rw0[e, pl.ds(j * 16, 16)] = jnp.zeros((16,), jnp.float32)
            return carry
        lax.fori_loop(0, _B, zero_grp, 0)
        for i in range(_RPT // _B):
            pltpu.async_copy(rw0, acc.at[pl.ds(r0 + i * _B, _B)], zsem)
        _ztail = _RPT - (_RPT // _B) * _B
        pltpu.async_copy(rw0.at[pl.ds(0, _ztail)],
                         acc.at[pl.ds(r0 + (_RPT // _B) * _B, _ztail)], zsem)
        for i in range(_RPT // _B):
            pltpu.make_async_copy(rw0, acc.at[pl.ds(r0 + i * _B, _B)],
                                  zsem).wait()
        pltpu.make_async_copy(rw0.at[pl.ds(0, _ztail)],
                              acc.at[pl.ds(r0 + (_RPT // _B) * _B, _ztail)],
                              zsem).wait()
        meta_wait(0, 0)
        start_gather(0, 0, 0)
        start_gather(0, 1, 1)
        plsc.subcore_barrier()

        def outer(mm, carry):
            for q in range(3):
                m = 3 * mm + q
                for b in range(3):
                    if q == 0 and b == 0:
                        @pl.when(mm > 0)
                        def _():
                            wait_scatter(2)
                    else:
                        wait_scatter((b - 1) % 3)
                    if b == 0:
                        if q == 0:
                            meta_fetch(m + 2, 2)
                        else:
                            @pl.when(mm < _MM - 1)
                            def _():
                                meta_fetch(m + 2, (q + 2) % 3)
                        if q < 2:
                            meta_wait(m + 1, q + 1)
                        else:
                            @pl.when(mm < _MM - 1)
                            def _():
                                meta_wait(m + 1, 0)
                    ch = q if b == 0 else (q + 1) % 3
                    bb = (b + 2) % 3
                    bg = (b + 2) % 3
                    if b == 0 or q < 2:
                        start_gather(ch, bb, bg)
                    else:
                        @pl.when(mm < _MM - 1)
                        def _():
                            start_gather(ch, bb, bg)
                    wait_gather(b)
                    scale(b, q)
                    for i in range(_B // 16):
                        sl = pl.ds(i * 16, 16)
                        didxs[b][sl] = dst_rs[q][pl.ds(b * _B + i * 16, 16)]
                    pltpu.async_copy(rowss[b], acc.at[didxs[b]],
                                     ssem, add=True)
            return carry

        lax.fori_loop(0, _MM, outer, 0)
        wait_scatter(2)
        plsc.subcore_barrier()
        pltpu.sync_copy(acc.at[pl.ds(r0, _RPT)],
                        out_hbm.at[c, pl.ds(r0, _RPT)])

    return body(x2, srcp, dstp, normp)


def _mm_body(a_ref, w_ref, o_ref):
    o_ref[...] = (
        jnp.dot(a_ref[0], w_ref[:_DH, :], preferred_element_type=jnp.float32)
        + jnp.dot(a_ref[1], w_ref[_DH:, :], preferred_element_type=jnp.float32)
    )


def _tc_matmul(a2, W):
    rows = 2000
    return pl.pallas_call(
        _mm_body,
        grid=(_N // rows,),
        in_specs=[
            pl.BlockSpec((2, rows, _DH), lambda i: (0, i, 0)),
            pl.BlockSpec((_D, _D), lambda i: (0, 0)),
        ],
        out_specs=pl.BlockSpec((rows, _D), lambda i: (i, 0)),
        out_shape=jax.ShapeDtypeStruct((_N, _D), jnp.float32),
    )(a2, W)


def kernel(x, r, W, edge_norm, edge_index, edge_type):
    del r, edge_type
    pad = _EPTP - _E // _NT
    src = edge_index[1].astype(jnp.int32).reshape(_NT, _E // _NT)
    dst = edge_index[0].astype(jnp.int32).reshape(_NT, _E // _NT)
    nrm = edge_norm.reshape(_NT, _E // _NT)
    srcp = jnp.pad(src, ((0, 0), (0, pad))).reshape(-1)
    dstp = jnp.pad(dst, ((0, 0), (0, pad))).reshape(-1)
    normp = jnp.pad(nrm, ((0, 0), (0, pad))).reshape(-1)
    x2 = x.reshape(2 * _N, _DH)
    a2 = _sc_aggregate(x2, srcp, dstp, normp)
    return _tc_matmul(a2, W)


# speedup vs baseline: 2.1721x; 2.1721x over previous
import functools

import jax
import jax.numpy as jnp
from jax import lax
from jax.experimental import pallas as pl
from jax.experimental.pallas import tpu as pltpu
from jax.experimental.pallas import tpu_sc as plsc

_N = 10000
_E = 160000
_D = 256
_DH = _D // 2
_NT = 16
_B = 96
_EPTP = 10368
_NBAT = _EPTP // _B
_CH = 3
_NCHUNK = _NBAT // _CH
_CB = _CH * _B
_MM = _NCHUNK // 3
_NBUF = 3
_NP = 10240
_RPT = _NP // _NT


def _sc_aggregate(x2, srcp, dstp, normp):
    mesh = plsc.VectorSubcoreMesh(core_axis_name="c", subcore_axis_name="s")

    @functools.partial(
        pl.kernel,
        mesh=mesh,
        out_type=jax.ShapeDtypeStruct((2, _NP, _DH), jnp.float32),
        scratch_types=(
            [pltpu.VMEM((_CB,), jnp.int32)] * 3
            + [pltpu.VMEM((_CB,), jnp.int32)] * 3
            + [pltpu.VMEM((_CB,), jnp.float32)] * 3
            + [pltpu.VMEM((_B,), jnp.int32)] * 3
            + [pltpu.VMEM((_B,), jnp.int32)] * 3
            + [pltpu.VMEM((_B, _DH), jnp.float32)] * 3
            + [pltpu.VMEM_SHARED((_NP, _DH), jnp.float32)]
            + [pltpu.SemaphoreType.DMA] * 8
        ),
    )
    def body(x2_hbm, src_hbm, dst_hbm, norm_hbm, out_hbm,
             sr0, sr1, sr2, dr0, dr1, dr2, nr0, nr1, nr2,
             di0, di1, di2, gi0, gi1, gi2, rw0, rw1, rw2, acc,
             ms0, ms1, ms2, gs0, gs1, gs2, ssem, zsem):
        src_rs = [sr0, sr1, sr2]
        dst_rs = [dr0, dr1, dr2]
        norm_rs = [nr0, nr1, nr2]
        didxs = [di0, di1, di2]
        gidxs = [gi0, gi1, gi2]
        rowss = [rw0, rw1, rw2]
        msems = [ms0, ms1, ms2]
        gsems = [gs0, gs1, gs2]
        c = lax.axis_index("c")
        s = lax.axis_index("s")
        r0 = s * _RPT
        e0 = s * _EPTP

        def meta_fetch(j, slot):
            off = e0 + j * _CB
            pltpu.async_copy(src_hbm.at[pl.ds(off, _CB)], src_rs[slot],
                             msems[slot])
            pltpu.async_copy(dst_hbm.at[pl.ds(off, _CB)], dst_rs[slot],
                             msems[slot])
            pltpu.async_copy(norm_hbm.at[pl.ds(off, _CB)], norm_rs[slot],
                             msems[slot])

        def meta_wait(j, slot):
            off = e0 + j * _CB
            pltpu.make_async_copy(src_hbm.at[pl.ds(off, _CB)],
                                  src_rs[slot], msems[slot]).wait()
            pltpu.make_async_copy(dst_hbm.at[pl.ds(off, _CB)],
                                  dst_rs[slot], msems[slot]).wait()
            pltpu.make_async_copy(norm_hbm.at[pl.ds(off, _CB)],
                                  norm_rs[slot], msems[slot]).wait()

        def start_gather(ch, bb, bg):
            for i in range(_B // 16):
                sl = pl.ds(i * 16, 16)
                gidxs[bg][sl] = src_rs[ch][pl.ds(bb * _B + i * 16, 16)] * 2 + c
            pltpu.async_copy(x2_hbm.at[gidxs[bg]], rowss[bg], gsems[bg])

        def wait_gather(b):
            pltpu.make_async_copy(x2_hbm.at[gidxs[b]], rowss[b],
                                  gsems[b]).wait()

        def wait_scatter(b):
            pltpu.make_async_copy(rowss[b], acc.at[didxs[b]], ssem).wait()

        def scale(b, q):
            def grp(g, inner):
                nv = norm_rs[q][pl.ds(b * _B + g * 16, 16)]
                for p in range(16):
                    e = g * 16 + p
                    nb = nv[p]
                    for j in range(_DH // 16):
                        sl = pl.ds(j * 16, 16)
                        rowss[b][e, sl] = rowss[b][e, sl] * nb
                return inner
            lax.fori_loop(0, _B // 16, grp, 0)

        meta_fetch(0, 0)
        meta_fetch(1, 1)
        def zero_grp(e, carry):
            for j in range(_DH // 16):
                rw0[e, pl.ds(j * 16, 16)] = jnp.zeros((16,), jnp.float32)
            return carry
        lax.fori_loop(0, _B, zero_grp, 0)
        for i in range(_RPT // _B):
            pltpu.async_copy(rw0, acc.at[pl.ds(r0 + i * _B, _B)], zsem)
        _ztail = _RPT - (_RPT // _B) * _B
        pltpu.async_copy(rw0.at[pl.ds(0, _ztail)],
                         acc.at[pl.ds(r0 + (_RPT // _B) * _B, _ztail)], zsem)
        for i in range(_RPT // _B):
            pltpu.make_async_copy(rw0, acc.at[pl.ds(r0 + i * _B, _B)],
                                  zsem).wait()
        pltpu.make_async_copy(rw0.at[pl.ds(0, _ztail)],
                              acc.at[pl.ds(r0 + (_RPT // _B) * _B, _ztail)],
                              zsem).wait()
        meta_wait(0, 0)
        start_gather(0, 0, 0)
        start_gather(0, 1, 1)
        plsc.subcore_barrier()

        def outer(mm, carry):
            for q in range(3):
                m = 3 * mm + q
                for b in range(3):
                    if q == 0 and b == 0:
                        @pl.when(mm > 0)
                        def _():
                            wait_scatter(2)
                    else:
                        wait_scatter((b - 1) % 3)
                    if b == 0:
                        if q == 0:
                            meta_fetch(m + 2, 2)
                        else:
                            @pl.when(mm < _MM - 1)
                            def _():
                                meta_fetch(m + 2, (q + 2) % 3)
                        if q < 2:
                            meta_wait(m + 1, q + 1)
                        else:
                            @pl.when(mm < _MM - 1)
                            def _():
                                meta_wait(m + 1, 0)
                    ch = q if b == 0 else (q + 1) % 3
                    bb = (b + 2) % 3
                    bg = (b + 2) % 3
                    if b == 0 or q < 2:
                        start_gather(ch, bb, bg)
                    else:
                        @pl.when(mm < _MM - 1)
                        def _():
                            start_gather(ch, bb, bg)
                    wait_gather(b)
                    scale(b, q)
                    for i in range(_B // 16):
                        sl = pl.ds(i * 16, 16)
                        didxs[b][sl] = dst_rs[q][pl.ds(b * _B + i * 16, 16)]
                    pltpu.async_copy(rowss[b], acc.at[didxs[b]],
                                     ssem, add=True)
            return carry

        lax.fori_loop(0, _MM, outer, 0)
        wait_scatter(2)
        plsc.subcore_barrier()
        pltpu.sync_copy(acc.at[pl.ds(r0, _RPT)],
                        out_hbm.at[c, pl.ds(r0, _RPT)])

    return body(x2, srcp, dstp, normp)


def _mm_body(a_ref, w_ref, o_ref):
    o_ref[...] = (
        jnp.dot(a_ref[0], w_ref[:_DH, :], preferred_element_type=jnp.float32)
        + jnp.dot(a_ref[1], w_ref[_DH:, :], preferred_element_type=jnp.float32)
    )


def _tc_matmul(a2, W):
    rows = 2000
    return pl.pallas_call(
        _mm_body,
        grid=(_N // rows,),
        in_specs=[
            pl.BlockSpec((2, rows, _DH), lambda i: (0, i, 0)),
            pl.BlockSpec((_D, _D), lambda i: (0, 0)),
        ],
        out_specs=pl.BlockSpec((rows, _D), lambda i: (i, 0)),
        out_shape=jax.ShapeDtypeStruct((_N, _D), jnp.float32),
    )(a2, W)


def kernel(x, r, W, edge_norm, edge_index, edge_type):
    del r, edge_type
    pad = _EPTP - _E // _NT
    src = edge_index[1].astype(jnp.int32).reshape(_NT, _E // _NT)
    dst = edge_index[0].astype(jnp.int32).reshape(_NT, _E // _NT)
    nrm = edge_norm.reshape(_NT, _E // _NT)
    spread = (jnp.arange(_NT * pad, dtype=jnp.int32) % _N).reshape(_NT, pad)
    srcp = jnp.concatenate([src, spread], axis=1).reshape(-1)
    dstp = jnp.pad(dst, ((0, 0), (0, pad))).reshape(-1)
    normp = jnp.pad(nrm, ((0, 0), (0, pad))).reshape(-1)
    x2 = x.reshape(2 * _N, _DH)
    a2 = _sc_aggregate(x2, srcp, dstp, normp)
    return _tc_matmul(a2, W)


# speedup vs baseline: 2.2458x; 1.0339x over previous
import functools

import jax
import jax.numpy as jnp
from jax import lax
from jax.experimental import pallas as pl
from jax.experimental.pallas import tpu as pltpu
from jax.experimental.pallas import tpu_sc as plsc

_N = 10000
_E = 160000
_D = 256
_DH = _D // 2
_NT = 16
_B = 96
_EPTP = 10368
_NBAT = _EPTP // _B
_CH = 3
_NCHUNK = _NBAT // _CH
_CB = _CH * _B
_MM = _NCHUNK // 3
_NBUF = 3
_NP = 10240
_RPT = _NP // _NT


def _sc_aggregate(x2, srcp, dstp, normp):
    mesh = plsc.VectorSubcoreMesh(core_axis_name="c", subcore_axis_name="s")

    @functools.partial(
        pl.kernel,
        mesh=mesh,
        out_type=jax.ShapeDtypeStruct((2, _NP, _DH), jnp.float32),
        scratch_types=(
            [pltpu.VMEM((_CB,), jnp.int32)] * 3
            + [pltpu.VMEM((_CB,), jnp.int32)] * 3
            + [pltpu.VMEM((_CB,), jnp.float32)] * 3
            + [pltpu.VMEM((_B,), jnp.int32)] * 3
            + [pltpu.VMEM((_B,), jnp.int32)] * 3
            + [pltpu.VMEM((_B, _DH), jnp.float32)] * 3
            + [pltpu.VMEM_SHARED((_NP, _DH), jnp.float32)]
            + [pltpu.SemaphoreType.DMA] * 8
        ),
    )
    def body(x2_hbm, src_hbm, dst_hbm, norm_hbm, out_hbm,
             sr0, sr1, sr2, dr0, dr1, dr2, nr0, nr1, nr2,
             di0, di1, di2, gi0, gi1, gi2, rw0, rw1, rw2, acc,
             ms0, ms1, ms2, gs0, gs1, gs2, ssem, zsem):
        src_rs = [sr0, sr1, sr2]
        dst_rs = [dr0, dr1, dr2]
        norm_rs = [nr0, nr1, nr2]
        didxs = [di0, di1, di2]
        gidxs = [gi0, gi1, gi2]
        rowss = [rw0, rw1, rw2]
        msems = [ms0, ms1, ms2]
        gsems = [gs0, gs1, gs2]
        c = lax.axis_index("c")
        s = lax.axis_index("s")
        r0 = s * _RPT
        e0 = s * _EPTP

        def meta_fetch(j, slot):
            off = e0 + j * _CB
            pltpu.async_copy(src_hbm.at[pl.ds(off, _CB)], src_rs[slot],
                             msems[slot])
            pltpu.async_copy(dst_hbm.at[pl.ds(off, _CB)], dst_rs[slot],
                             msems[slot])
            pltpu.async_copy(norm_hbm.at[pl.ds(off, _CB)], norm_rs[slot],
                             msems[slot])

        def meta_wait(j, slot):
            off = e0 + j * _CB
            pltpu.make_async_copy(src_hbm.at[pl.ds(off, _CB)],
                                  src_rs[slot], msems[slot]).wait()
            pltpu.make_async_copy(dst_hbm.at[pl.ds(off, _CB)],
                                  dst_rs[slot], msems[slot]).wait()
            pltpu.make_async_copy(norm_hbm.at[pl.ds(off, _CB)],
                                  norm_rs[slot], msems[slot]).wait()

        def start_gather(ch, bb, bg):
            for i in range(_B // 16):
                sl = pl.ds(i * 16, 16)
                gidxs[bg][sl] = src_rs[ch][pl.ds(bb * _B + i * 16, 16)] * 2 + c
            pltpu.async_copy(x2_hbm.at[gidxs[bg]], rowss[bg], gsems[bg])

        def wait_gather(b):
            pltpu.make_async_copy(x2_hbm.at[gidxs[b]], rowss[b],
                                  gsems[b]).wait()

        def wait_scatter(b):
            pltpu.make_async_copy(rowss[b], acc.at[didxs[b]], ssem).wait()

        def scale(b, q):
            def grp(g, inner):
                nv = norm_rs[q][pl.ds(b * _B + g * 16, 16)]
                for p in range(16):
                    e = g * 16 + p
                    nb = nv[p]
                    for j in range(_DH // 16):
                        sl = pl.ds(j * 16, 16)
                        rowss[b][e, sl] = rowss[b][e, sl] * nb
                return inner
            lax.fori_loop(0, _B // 16, grp, 0)

        meta_fetch(0, 0)
        meta_fetch(1, 1)
        def zero_grp(e, carry):
            for j in range(_DH // 16):
                rw0[e, pl.ds(j * 16, 16)] = jnp.zeros((16,), jnp.float32)
            return carry
        lax.fori_loop(0, _B, zero_grp, 0)
        for i in range(_RPT // _B):
            pltpu.async_copy(rw0, acc.at[pl.ds(r0 + i * _B, _B)], zsem)
        _ztail = _RPT - (_RPT // _B) * _B
        pltpu.async_copy(rw0.at[pl.ds(0, _ztail)],
                         acc.at[pl.ds(r0 + (_RPT // _B) * _B, _ztail)], zsem)
        for i in range(_RPT // _B):
            pltpu.make_async_copy(rw0, acc.at[pl.ds(r0 + i * _B, _B)],
                                  zsem).wait()
        pltpu.make_async_copy(rw0.at[pl.ds(0, _ztail)],
                              acc.at[pl.ds(r0 + (_RPT // _B) * _B, _ztail)],
                              zsem).wait()
        meta_wait(0, 0)
        start_gather(0, 0, 0)
        start_gather(0, 1, 1)
        plsc.subcore_barrier()

        def outer(mm, carry):
            for q in range(3):
                m = 3 * mm + q
                for b in range(3):
                    wait_gather(b)
                    scale(b, q)
                    if q == 0 and b == 0:
                        @pl.when(mm > 0)
                        def _():
                            wait_scatter(2)
                    else:
                        wait_scatter((b - 1) % 3)
                    if b == 0:
                        if q == 0:
                            meta_fetch(m + 2, 2)
                        else:
                            @pl.when(mm < _MM - 1)
                            def _():
                                meta_fetch(m + 2, (q + 2) % 3)
                        if q < 2:
                            meta_wait(m + 1, q + 1)
                        else:
                            @pl.when(mm < _MM - 1)
                            def _():
                                meta_wait(m + 1, 0)
                    ch = q if b == 0 else (q + 1) % 3
                    bb = (b + 2) % 3
                    bg = (b + 2) % 3
                    if b == 0 or q < 2:
                        start_gather(ch, bb, bg)
                    else:
                        @pl.when(mm < _MM - 1)
                        def _():
                            start_gather(ch, bb, bg)
                    for i in range(_B // 16):
                        sl = pl.ds(i * 16, 16)
                        didxs[b][sl] = dst_rs[q][pl.ds(b * _B + i * 16, 16)]
                    pltpu.async_copy(rowss[b], acc.at[didxs[b]],
                                     ssem, add=True)
            return carry

        lax.fori_loop(0, _MM, outer, 0)
        wait_scatter(2)
        plsc.subcore_barrier()
        pltpu.sync_copy(acc.at[pl.ds(r0, _RPT)],
                        out_hbm.at[c, pl.ds(r0, _RPT)])

    return body(x2, srcp, dstp, normp)


def _mm_body(a_ref, w_ref, o_ref):
    o_ref[...] = (
        jnp.dot(a_ref[0], w_ref[:_DH, :], preferred_element_type=jnp.float32)
        + jnp.dot(a_ref[1], w_ref[_DH:, :], preferred_element_type=jnp.float32)
    )


def _tc_matmul(a2, W):
    rows = 2000
    return pl.pallas_call(
        _mm_body,
        grid=(_N // rows,),
        in_specs=[
            pl.BlockSpec((2, rows, _DH), lambda i: (0, i, 0)),
            pl.BlockSpec((_D, _D), lambda i: (0, 0)),
        ],
        out_specs=pl.BlockSpec((rows, _D), lambda i: (i, 0)),
        out_shape=jax.ShapeDtypeStruct((_N, _D), jnp.float32),
    )(a2, W)


def kernel(x, r, W, edge_norm, edge_index, edge_type):
    del r, edge_type
    pad = _EPTP - _E // _NT
    src = edge_index[1].astype(jnp.int32).reshape(_NT, _E // _NT)
    dst = edge_index[0].astype(jnp.int32).reshape(_NT, _E // _NT)
    nrm = edge_norm.reshape(_NT, _E // _NT)
    spread = (jnp.arange(_NT * pad, dtype=jnp.int32) % _N).reshape(_NT, pad)
    srcp = jnp.concatenate([src, spread], axis=1).reshape(-1)
    dstp = jnp.pad(dst, ((0, 0), (0, pad))).reshape(-1)
    normp = jnp.pad(nrm, ((0, 0), (0, pad))).reshape(-1)
    x2 = x.reshape(2 * _N, _DH)
    a2 = _sc_aggregate(x2, srcp, dstp, normp)
    return _tc_matmul(a2, W)
